# baseline (speedup 1.0000x reference)
import functools
import math

import jax
import jax.numpy as jnp
from jax import lax
from jax.experimental import pallas as pl
from jax.experimental.pallas import tpu as pltpu
from jax.experimental.pallas import tpu_sc as plsc

N = 10000
C = 128
E = 320000
S = 64
H = 16
DH = C // H


NC = 2
NS = 16
NW = NC * NS
EPW = E // NW
KCH = 80
NCHUNK = EPW // KCH
NSLOT = 4
NPAD = 10240
RPT = NPAD // NS
ZR = 32


def _sc_agg_body(x_hbm, col_hbm, row_hbm, val_hbm, out_hbm,
                 colvs, valvs, rowvs, bufs, zv, acc,
                 gsems, ssems, lsems, csems):
    cid = lax.axis_index("c")
    sid = lax.axis_index("s")
    wid = cid * NS + sid

    def zero_body(i, carry):
        for j in range(C // 16):
            zv[i, pl.ds(j * 16, 16)] = jnp.zeros((16,), jnp.float32)
        return carry
    lax.fori_loop(0, ZR, zero_body, 0)
    for t in range(RPT // ZR):
        pltpu.sync_copy(zv, acc.at[pl.ds(sid * RPT + t * ZR, ZR), :])

    plsc.subcore_barrier()

    def load_col(ci, k):
        pltpu.async_copy(col_hbm.at[pl.ds(wid * EPW + ci * KCH, KCH)],
                         colvs[k], csems[k])

    def start_chunk(ci, k):
        base = wid * EPW + ci * KCH
        pltpu.async_copy(row_hbm.at[pl.ds(base, KCH)], rowvs[k], lsems[k])
        pltpu.async_copy(val_hbm.at[pl.ds(base, KCH)], valvs[k], lsems[k])
        pltpu.make_async_copy(col_hbm.at[pl.ds(0, KCH)],
                              colvs[k], csems[k]).wait()
        pltpu.async_copy(x_hbm.at[colvs[k]], bufs[k], gsems[k])

    def wait_gather(k):
        pltpu.make_async_copy(x_hbm.at[colvs[k]], bufs[k], gsems[k]).wait()

    def wait_rows(k):
        pltpu.make_async_copy(row_hbm.at[pl.ds(0, KCH)],
                              rowvs[k], lsems[k]).wait()
        pltpu.make_async_copy(val_hbm.at[pl.ds(0, KCH)],
                              valvs[k], lsems[k]).wait()

    def scale(k):
        buf, vref = bufs[k], valvs[k]

        def one_edge(e, vv):
            for j in range(C // 16):
                r = buf[e, pl.ds(j * 16, 16)]
                buf[e, pl.ds(j * 16, 16)] = r * vv

        def edge_body(i, c2):
            e = 2 * i
            vv0 = plsc.load_gather(vref, [jnp.full((16,), e, jnp.int32)])
            vv1 = plsc.load_gather(vref, [jnp.full((16,), e + 1, jnp.int32)])
            one_edge(e, vv0)
            one_edge(e + 1, vv1)
            return c2
        lax.fori_loop(0, KCH // 2, edge_body, 0)

    def start_scatter(k):
        pltpu.async_copy(bufs[k], acc.at[rowvs[k]], ssems[k], add=True)

    def wait_scatter(k):
        pltpu.make_async_copy(bufs[k], acc.at[rowvs[k]], ssems[k]).wait()

    def step(c, k, issue_next=True, wait_sc=True, issue_col=True):
        if wait_sc:
            wait_scatter((k + 2) % NSLOT)
        if issue_next:
            start_chunk(c + 2, (k + 2) % NSLOT)
        wait_gather(k)
        if issue_col:
            load_col(c + NSLOT, k)
        scale(k)
        wait_rows(k)
        start_scatter(k)

    for k in range(NSLOT):
        load_col(k, k)
    start_chunk(0, 0)
    start_chunk(1, 1)
    step(0, 0, wait_sc=False)
    step(1, 1, wait_sc=False)

    def quad_body(t, carry):
        c = 4 * t + 2
        step(c, 2)
        step(c + 1, 3)
        step(c + 2, 0)
        step(c + 3, 1)
        return carry
    NQ = (NCHUNK - 6) // 4
    lax.fori_loop(0, NQ, quad_body, 0)

    cb = 4 * NQ + 2
    for c in range(cb, NCHUNK):
        step(c, c % NSLOT,
             issue_next=(c + 2 < NCHUNK),
             issue_col=(c + NSLOT < NCHUNK))
    wait_scatter((NCHUNK - 2) % NSLOT)
    wait_scatter((NCHUNK - 1) % NSLOT)

    plsc.subcore_barrier()
    pltpu.sync_copy(acc.at[pl.ds(sid * RPT, RPT), :],
                    out_hbm.at[cid, pl.ds(sid * RPT, RPT), :])


def _sc_aggregate(xs, col, row, vals):
    mesh = plsc.VectorSubcoreMesh(core_axis_name="c", subcore_axis_name="s")
    f = pl.kernel(
        _sc_agg_body,
        out_type=jax.ShapeDtypeStruct((NC, NPAD, C), jnp.float32),
        mesh=mesh,
        scratch_types=[
            [pltpu.VMEM((KCH,), jnp.int32) for _ in range(NSLOT)],
            [pltpu.VMEM((KCH,), jnp.float32)
             for _ in range(NSLOT)],
            [pltpu.VMEM((KCH,), jnp.int32) for _ in range(NSLOT)],
            [pltpu.VMEM((KCH, C), jnp.float32)
             for _ in range(NSLOT)],
            pltpu.VMEM((ZR, C), jnp.float32),
            pltpu.VMEM_SHARED((NPAD, C), jnp.float32),
            [pltpu.SemaphoreType.DMA for _ in range(NSLOT)],
            [pltpu.SemaphoreType.DMA for _ in range(NSLOT)],
            [pltpu.SemaphoreType.DMA for _ in range(NSLOT)],
            [pltpu.SemaphoreType.DMA for _ in range(NSLOT)],
        ],
        compiler_params=pltpu.CompilerParams(needs_layout_passes=False),
    )
    return f(xs, col, row, vals)


NB = 10
BN = N // NB

_INV_SQRT2 = 1.0 / math.sqrt(2.0)
_INV_SQRT_DH = 1.0 / math.sqrt(DH)


def _tc_fused_body(x_ref, g_ref, wasT_ref, bas_ref, wagT_ref, lng_ref,
                   lnb_ref, wspT_ref, bsp_ref, inpT_ref, inpb_ref, outT_ref,
                   outb_ref, out_ref, w_s, num_s, ws_s, so_s):
    i = pl.program_id(0)

    @pl.when(i < NB)
    def _():
        xb = x_ref[...]
        g = g_ref[0] + g_ref[1]
        a = (jnp.dot(xb, wasT_ref[...], preferred_element_type=jnp.float32)
             + bas_ref[...]
             + jnp.dot(g, wagT_ref[...], preferred_element_type=jnp.float32))
        mu = jnp.mean(a, axis=-1, keepdims=True)
        d = a - mu
        var = jnp.mean(d * d, axis=-1, keepdims=True)
        ln = d * lax.rsqrt(var + 1e-5) * lng_ref[...] + lnb_ref[...]
        ge = 0.5 * ln * (1.0 + lax.erf(ln * _INV_SQRT2))
        logits = jnp.dot(ge, wspT_ref[...],
                         preferred_element_type=jnp.float32) + bsp_ref[...]
        m = jnp.max(logits, axis=-1, keepdims=True)
        p = jnp.exp(logits - m)
        w = p / jnp.sum(p, axis=-1, keepdims=True)
        w_s[pl.ds(i * BN, BN), :] = w

        @pl.when(i == 0)
        def _():
            num_s[...] = jnp.zeros_like(num_s)
            ws_s[...] = jnp.zeros_like(ws_s)
        num_s[...] += lax.dot_general(
            w, xb, (((0,), (0,)), ((), ())),
            preferred_element_type=jnp.float32)
        ws_s[...] += lax.dot_general(
            w, jnp.ones((BN, 1), jnp.float32), (((0,), (0,)), ((), ())),
            preferred_element_type=jnp.float32)

    @pl.when(i == NB)
    def _():
        recip = 1.0 / jnp.maximum(ws_s[...], 1e-8)
        slices = num_s[...] * recip
        qkv = jnp.dot(slices, inpT_ref[...],
                      preferred_element_type=jnp.float32) + inpb_ref[...]
        q = qkv[:, :C]
        k = qkv[:, C:2 * C]
        v = qkv[:, 2 * C:]
        outs = []
        for h in range(H):
            qh = q[:, h * DH:(h + 1) * DH]
            kh = k[:, h * DH:(h + 1) * DH]
            vh = v[:, h * DH:(h + 1) * DH]
            s = lax.dot_general(
                qh, kh, (((1,), (1,)), ((), ())),
                preferred_element_type=jnp.float32) * _INV_SQRT_DH
            m = jnp.max(s, axis=-1, keepdims=True)
            p = jnp.exp(s - m)
            attn = p / jnp.sum(p, axis=-1, keepdims=True)
            outs.append(jnp.dot(attn, vh, preferred_element_type=jnp.float32))
        o = jnp.concatenate(outs, axis=1)
        so_s[...] = jnp.dot(o, outT_ref[...],
                            preferred_element_type=jnp.float32) + outb_ref[...]

    @pl.when(i > NB)
    def _():
        j = i - NB - 1
        out_ref[...] = jnp.dot(w_s[pl.ds(j * BN, BN), :], so_s[...],
                               preferred_element_type=jnp.float32)


_CT = (((1,), (1,)), ((), ()))


def _mha(num, ws, inpw, inpb, outw, outb):
    recip = 1.0 / jnp.maximum(ws, 1e-8)
    slices = num * recip
    qkv = lax.dot_general(slices, inpw, _CT,
                          preferred_element_type=jnp.float32) + inpb
    q = qkv[:, :C]
    k = qkv[:, C:2 * C]
    v = qkv[:, 2 * C:]
    outs = []
    for h in range(H):
        qh = q[:, h * DH:(h + 1) * DH]
        kh = k[:, h * DH:(h + 1) * DH]
        vh = v[:, h * DH:(h + 1) * DH]
        s = lax.dot_general(qh, kh, (((1,), (1,)), ((), ())),
                            preferred_element_type=jnp.float32) * _INV_SQRT_DH
        m = jnp.max(s, axis=-1, keepdims=True)
        p = jnp.exp(s - m)
        attn = p / jnp.sum(p, axis=-1, keepdims=True)
        outs.append(jnp.dot(attn, vh, preferred_element_type=jnp.float32))
    o = jnp.concatenate(outs, axis=1)
    return lax.dot_general(o, outw, _CT,
                           preferred_element_type=jnp.float32) + outb


def _tc_one_body(x_ref, g_ref, was_ref, bas_ref, wag_ref, lng_ref,
                 lnb_ref, wsp_ref, bsp_ref, inpw_ref, inpb_ref, outw_ref,
                 outb_ref, out_ref):
    xb = x_ref[...]
    g = (g_ref[0] + g_ref[1])[:N, :]
    a = (lax.dot_general(xb, was_ref[...], _CT,
                         preferred_element_type=jnp.float32)
         + bas_ref[...]
         + lax.dot_general(g, wag_ref[...], _CT,
                           preferred_element_type=jnp.float32))
    mu = jnp.mean(a, axis=-1, keepdims=True)
    d = a - mu
    var = jnp.mean(d * d, axis=-1, keepdims=True)
    ln = d * lax.rsqrt(var + 1e-5) * lng_ref[...] + lnb_ref[...]
    ge = 0.5 * ln * (1.0 + lax.erf(ln * _INV_SQRT2))
    logits = lax.dot_general(ge, wsp_ref[...], _CT,
                             preferred_element_type=jnp.float32) + bsp_ref[...]
    m = jnp.max(logits, axis=-1, keepdims=True)
    p = jnp.exp(logits - m)
    w = p / jnp.sum(p, axis=-1, keepdims=True)
    num = lax.dot_general(w, xb, (((0,), (0,)), ((), ())),
                          preferred_element_type=jnp.float32)
    ws = lax.dot_general(w, jnp.ones((N, 1), jnp.float32),
                         (((0,), (0,)), ((), ())),
                         preferred_element_type=jnp.float32)
    so = _mha(num, ws, inpw_ref[...], inpb_ref[...], outw_ref[...],
              outb_ref[...])
    out_ref[...] = jnp.dot(w, so, preferred_element_type=jnp.float32)


def _full(shape):
    return pl.BlockSpec(shape, lambda i: tuple(0 for _ in shape))


def kernel(x, adj_indices, adj_values, W_as, b_as, W_ag, ln_g, ln_b,
           W_sp, b_sp, in_proj_w, in_proj_b, out_w, out_b):
    xs = x.reshape(N, C)
    row = adj_indices[0]
    col = adj_indices[1]

    partials = _sc_aggregate(xs, col, row, adj_values)

    out = pl.pallas_call(
        _tc_one_body,
        out_shape=jax.ShapeDtypeStruct((N, C), jnp.float32),
        compiler_params=pltpu.CompilerParams(
            vmem_limit_bytes=120 * 1024 * 1024),
    )(xs, partials, W_as, b_as.reshape(1, C), W_ag,
      ln_g.reshape(1, C), ln_b.reshape(1, C), W_sp, b_sp.reshape(1, S),
      in_proj_w, in_proj_b.reshape(1, 3 * C), out_w, out_b.reshape(1, C))

    return out.reshape(1, N, C)


# speedup vs baseline: 1.0061x; 1.0061x over previous
import functools
import math

import jax
import jax.numpy as jnp
from jax import lax
from jax.experimental import pallas as pl
from jax.experimental.pallas import tpu as pltpu
from jax.experimental.pallas import tpu_sc as plsc

N = 10000
C = 128
E = 320000
S = 64
H = 16
DH = C // H


NC = 2
NS = 16
NW = NC * NS
EPW = E // NW
KCH = 80
NCHUNK = EPW // KCH
NSLOT = 4
NPAD = 10240
RPT = NPAD // NS
ZR = 32


def _sc_agg_body(x_hbm, col_hbm, row_hbm, val_hbm, out_hbm,
                 colvs, valvs, rowvs, bufs, zv, acc,
                 gsems, ssems, lsems, csems):
    cid = lax.axis_index("c")
    sid = lax.axis_index("s")
    wid = cid * NS + sid

    def zero_body(i, carry):
        for j in range(C // 16):
            zv[i, pl.ds(j * 16, 16)] = jnp.zeros((16,), jnp.float32)
        return carry
    lax.fori_loop(0, ZR, zero_body, 0)
    for t in range(RPT // ZR):
        pltpu.sync_copy(zv, acc.at[pl.ds(sid * RPT + t * ZR, ZR), :])

    plsc.subcore_barrier()

    def load_col(ci, k):
        pltpu.async_copy(col_hbm.at[pl.ds(wid * EPW + ci * KCH, KCH)],
                         colvs[k], csems[k])

    def start_chunk(ci, k):
        base = wid * EPW + ci * KCH
        pltpu.async_copy(row_hbm.at[pl.ds(base, KCH)], rowvs[k], lsems[k])
        pltpu.async_copy(val_hbm.at[pl.ds(base, KCH)], valvs[k], lsems[k])
        pltpu.make_async_copy(col_hbm.at[pl.ds(0, KCH)],
                              colvs[k], csems[k]).wait()
        pltpu.async_copy(x_hbm.at[colvs[k]], bufs[k], gsems[k])

    def wait_gather(k):
        pltpu.make_async_copy(x_hbm.at[colvs[k]], bufs[k], gsems[k]).wait()

    def wait_rows(k):
        pltpu.make_async_copy(row_hbm.at[pl.ds(0, KCH)],
                              rowvs[k], lsems[k]).wait()
        pltpu.make_async_copy(val_hbm.at[pl.ds(0, KCH)],
                              valvs[k], lsems[k]).wait()

    def scale(k):
        buf, vref = bufs[k], valvs[k]

        def one_edge(e, vv):
            for j in range(C // 16):
                r = buf[e, pl.ds(j * 16, 16)]
                buf[e, pl.ds(j * 16, 16)] = r * vv

        def edge_body(i, c2):
            e = 2 * i
            vv0 = plsc.load_gather(vref, [jnp.full((16,), e, jnp.int32)])
            vv1 = plsc.load_gather(vref, [jnp.full((16,), e + 1, jnp.int32)])
            one_edge(e, vv0)
            one_edge(e + 1, vv1)
            return c2
        lax.fori_loop(0, KCH // 2, edge_body, 0)

    def start_scatter(k):
        pltpu.async_copy(bufs[k], acc.at[rowvs[k]], ssems[k], add=True)

    def wait_scatter(k):
        pltpu.make_async_copy(bufs[k], acc.at[rowvs[k]], ssems[k]).wait()

    def step(c, k, issue_next=True, wait_sc=True, issue_col=True):
        if wait_sc:
            wait_scatter((k + 2) % NSLOT)
        if issue_next:
            start_chunk(c + 2, (k + 2) % NSLOT)
        wait_gather(k)
        if issue_col:
            load_col(c + NSLOT, k)
        scale(k)
        wait_rows(k)
        start_scatter(k)

    for k in range(NSLOT):
        load_col(k, k)
    start_chunk(0, 0)
    start_chunk(1, 1)
    step(0, 0, wait_sc=False)
    step(1, 1, wait_sc=False)

    def quad_body(t, carry):
        c = 4 * t + 2
        step(c, 2)
        step(c + 1, 3)
        step(c + 2, 0)
        step(c + 3, 1)
        return carry
    NQ = (NCHUNK - 6) // 4
    lax.fori_loop(0, NQ, quad_body, 0)

    cb = 4 * NQ + 2
    for c in range(cb, NCHUNK):
        step(c, c % NSLOT,
             issue_next=(c + 2 < NCHUNK),
             issue_col=(c + NSLOT < NCHUNK))
    wait_scatter((NCHUNK - 2) % NSLOT)
    wait_scatter((NCHUNK - 1) % NSLOT)

    plsc.subcore_barrier()
    pltpu.sync_copy(acc.at[pl.ds(sid * RPT, RPT), :],
                    out_hbm.at[cid, pl.ds(sid * RPT, RPT), :])


def _sc_aggregate(xs, col, row, vals):
    mesh = plsc.VectorSubcoreMesh(core_axis_name="c", subcore_axis_name="s")
    f = pl.kernel(
        _sc_agg_body,
        out_type=jax.ShapeDtypeStruct((NC, NPAD, C), jnp.float32),
        mesh=mesh,
        scratch_types=[
            [pltpu.VMEM((KCH,), jnp.int32) for _ in range(NSLOT)],
            [pltpu.VMEM((KCH,), jnp.float32)
             for _ in range(NSLOT)],
            [pltpu.VMEM((KCH,), jnp.int32) for _ in range(NSLOT)],
            [pltpu.VMEM((KCH, C), jnp.float32)
             for _ in range(NSLOT)],
            pltpu.VMEM((ZR, C), jnp.float32),
            pltpu.VMEM_SHARED((NPAD, C), jnp.float32),
            [pltpu.SemaphoreType.DMA for _ in range(NSLOT)],
            [pltpu.SemaphoreType.DMA for _ in range(NSLOT)],
            [pltpu.SemaphoreType.DMA for _ in range(NSLOT)],
            [pltpu.SemaphoreType.DMA for _ in range(NSLOT)],
        ],
        compiler_params=pltpu.CompilerParams(needs_layout_passes=False),
    )
    return f(xs, col, row, vals)


NB = 5
BN = N // NB

_INV_SQRT2 = 1.0 / math.sqrt(2.0)
_INV_SQRT_DH = 1.0 / math.sqrt(DH)


def _tc_fused_body(x_ref, g_ref, wasT_ref, bas_ref, wagT_ref, lng_ref,
                   lnb_ref, wspT_ref, bsp_ref, inpT_ref, inpb_ref, outT_ref,
                   outb_ref, out_ref, w_s, num_s, ws_s, so_s):
    i = pl.program_id(0)

    @pl.when(i < NB)
    def _():
        xb = x_ref[...]
        g = g_ref[0] + g_ref[1]
        a = (jnp.dot(xb, wasT_ref[...], preferred_element_type=jnp.float32)
             + bas_ref[...]
             + jnp.dot(g, wagT_ref[...], preferred_element_type=jnp.float32))
        mu = jnp.mean(a, axis=-1, keepdims=True)
        d = a - mu
        var = jnp.mean(d * d, axis=-1, keepdims=True)
        ln = d * lax.rsqrt(var + 1e-5) * lng_ref[...] + lnb_ref[...]
        ge = 0.5 * ln * (1.0 + lax.erf(ln * _INV_SQRT2))
        logits = jnp.dot(ge, wspT_ref[...],
                         preferred_element_type=jnp.float32) + bsp_ref[...]
        m = jnp.max(logits, axis=-1, keepdims=True)
        p = jnp.exp(logits - m)
        w = p / jnp.sum(p, axis=-1, keepdims=True)
        w_s[pl.ds(i * BN, BN), :] = w

        @pl.when(i == 0)
        def _():
            num_s[...] = jnp.zeros_like(num_s)
            ws_s[...] = jnp.zeros_like(ws_s)
        num_s[...] += lax.dot_general(
            w, xb, (((0,), (0,)), ((), ())),
            preferred_element_type=jnp.float32)
        ws_s[...] += lax.dot_general(
            w, jnp.ones((BN, 1), jnp.float32), (((0,), (0,)), ((), ())),
            preferred_element_type=jnp.float32)

    @pl.when(i == NB)
    def _():
        recip = 1.0 / jnp.maximum(ws_s[...], 1e-8)
        slices = num_s[...] * recip
        qkv = jnp.dot(slices, inpT_ref[...],
                      preferred_element_type=jnp.float32) + inpb_ref[...]
        q = qkv[:, :C]
        k = qkv[:, C:2 * C]
        v = qkv[:, 2 * C:]
        outs = []
        for h in range(H):
            qh = q[:, h * DH:(h + 1) * DH]
            kh = k[:, h * DH:(h + 1) * DH]
            vh = v[:, h * DH:(h + 1) * DH]
            s = lax.dot_general(
                qh, kh, (((1,), (1,)), ((), ())),
                preferred_element_type=jnp.float32) * _INV_SQRT_DH
            m = jnp.max(s, axis=-1, keepdims=True)
            p = jnp.exp(s - m)
            attn = p / jnp.sum(p, axis=-1, keepdims=True)
            outs.append(jnp.dot(attn, vh, preferred_element_type=jnp.float32))
        o = jnp.concatenate(outs, axis=1)
        so_s[...] = jnp.dot(o, outT_ref[...],
                            preferred_element_type=jnp.float32) + outb_ref[...]

    @pl.when(i > NB)
    def _():
        j = i - NB - 1
        out_ref[...] = jnp.dot(w_s[pl.ds(j * BN, BN), :], so_s[...],
                               preferred_element_type=jnp.float32)


_CT = (((1,), (1,)), ((), ()))


def _mha(num, ws, inpw, inpb, outw, outb):
    recip = 1.0 / jnp.maximum(ws, 1e-8)
    slices = num * recip
    qkv = lax.dot_general(slices, inpw, _CT,
                          preferred_element_type=jnp.float32) + inpb
    q = qkv[:, :C]
    k = qkv[:, C:2 * C]
    v = qkv[:, 2 * C:]
    outs = []
    for h in range(H):
        qh = q[:, h * DH:(h + 1) * DH]
        kh = k[:, h * DH:(h + 1) * DH]
        vh = v[:, h * DH:(h + 1) * DH]
        s = lax.dot_general(qh, kh, (((1,), (1,)), ((), ())),
                            preferred_element_type=jnp.float32) * _INV_SQRT_DH
        m = jnp.max(s, axis=-1, keepdims=True)
        p = jnp.exp(s - m)
        attn = p / jnp.sum(p, axis=-1, keepdims=True)
        outs.append(jnp.dot(attn, vh, preferred_element_type=jnp.float32))
    o = jnp.concatenate(outs, axis=1)
    return lax.dot_general(o, outw, _CT,
                           preferred_element_type=jnp.float32) + outb


def _tc_one_body(x_ref, g_ref, was_ref, bas_ref, wag_ref, lng_ref,
                 lnb_ref, wsp_ref, bsp_ref, inpw_ref, inpb_ref, outw_ref,
                 outb_ref, out_ref):
    xb = x_ref[...]
    g = (g_ref[0] + g_ref[1])[:N, :]
    a = (lax.dot_general(xb, was_ref[...], _CT,
                         preferred_element_type=jnp.float32)
         + bas_ref[...]
         + lax.dot_general(g, wag_ref[...], _CT,
                           preferred_element_type=jnp.float32))
    mu = jnp.mean(a, axis=-1, keepdims=True)
    d = a - mu
    var = jnp.mean(d * d, axis=-1, keepdims=True)
    ln = d * lax.rsqrt(var + 1e-5) * lng_ref[...] + lnb_ref[...]
    ge = 0.5 * ln * (1.0 + lax.erf(ln * _INV_SQRT2))
    logits = lax.dot_general(ge, wsp_ref[...], _CT,
                             preferred_element_type=jnp.float32) + bsp_ref[...]
    m = jnp.max(logits, axis=-1, keepdims=True)
    p = jnp.exp(logits - m)
    w = p / jnp.sum(p, axis=-1, keepdims=True)
    num = lax.dot_general(w, xb, (((0,), (0,)), ((), ())),
                          preferred_element_type=jnp.float32)
    ws = lax.dot_general(w, jnp.ones((N, 1), jnp.float32),
                         (((0,), (0,)), ((), ())),
                         preferred_element_type=jnp.float32)
    so = _mha(num, ws, inpw_ref[...], inpb_ref[...], outw_ref[...],
              outb_ref[...])
    out_ref[...] = jnp.dot(w, so, preferred_element_type=jnp.float32)


def _full(shape):
    return pl.BlockSpec(shape, lambda i: tuple(0 for _ in shape))


def kernel(x, adj_indices, adj_values, W_as, b_as, W_ag, ln_g, ln_b,
           W_sp, b_sp, in_proj_w, in_proj_b, out_w, out_b):
    xs = x.reshape(N, C)
    row = adj_indices[0]
    col = adj_indices[1]

    partials = _sc_aggregate(xs, col, row, adj_values)

    nbm = NB - 1
    out = pl.pallas_call(
        _tc_fused_body,
        grid=(2 * NB + 1,),
        in_specs=[
            pl.BlockSpec((BN, C), lambda i: (jnp.minimum(i, nbm), 0)),
            pl.BlockSpec((2, BN, C), lambda i: (0, jnp.minimum(i, nbm), 0)),
            _full((C, C)),
            _full((1, C)),
            _full((C, C)),
            _full((1, C)),
            _full((1, C)),
            _full((C, S)),
            _full((1, S)),
            _full((C, 3 * C)),
            _full((1, 3 * C)),
            _full((C, C)),
            _full((1, C)),
        ],
        out_specs=pl.BlockSpec(
            (BN, C), lambda i: (jnp.maximum(i - NB - 1, 0), 0)),
        out_shape=jax.ShapeDtypeStruct((N, C), jnp.float32),
        scratch_shapes=[
            pltpu.VMEM((N, S), jnp.float32),
            pltpu.VMEM((S, C), jnp.float32),
            pltpu.VMEM((S, 1), jnp.float32),
            pltpu.VMEM((S, C), jnp.float32),
        ],
        compiler_params=pltpu.CompilerParams(
            vmem_limit_bytes=120 * 1024 * 1024),
    )(xs, partials, W_as.T, b_as.reshape(1, C), W_ag.T,
      ln_g.reshape(1, C), ln_b.reshape(1, C), W_sp.T, b_sp.reshape(1, S),
      in_proj_w.T, in_proj_b.reshape(1, 3 * C), out_w.T, out_b.reshape(1, C))

    return out.reshape(1, N, C)
